# Initial kernel scaffold; baseline (speedup 1.0000x reference)
import jax
import jax.numpy as jnp
from jax.experimental import pallas as pl


def kernel(embeddings, targets):
    raise NotImplementedError("write your pallas kernel here")


# speedup vs baseline: 1.8368x; 1.8368x over previous
import functools

import jax
import jax.numpy as jnp
from jax.experimental import pallas as pl

MARGIN_ = 1.0


def _triplet_block(e_blk_ref, e_full_ref, t_ref, loss_ref, cnt_ref, acc_ref,
                   *, blk: int, batch: int):
    i = pl.program_id(0)

    eb = e_blk_ref[...]
    ef = e_full_ref[...]

    sq_b = jnp.sum(eb * eb, axis=1)
    sq_f = jnp.sum(ef * ef, axis=1)
    gram = jax.lax.dot_general(
        eb, ef, (((1,), (1,)), ((), ())),
        preferred_element_type=jnp.float32)
    s = sq_b[:, None] + sq_f[None, :] - 2.0 * gram
    s = jnp.maximum(s, 0.0)

    t_full = t_ref[0, :]
    t_blk = t_ref[0, pl.ds(i * blk, blk)]
    same = t_blk[:, None] == t_full[None, :]

    rows = i * blk + jax.lax.broadcasted_iota(jnp.int32, (blk, batch), 0)
    cols = jax.lax.broadcasted_iota(jnp.int32, (blk, batch), 1)
    upper = rows < cols

    neg_s = jnp.where(same, jnp.inf, s)
    min_neg_s = jnp.min(neg_s, axis=1)
    min_neg_d = jnp.sqrt(min_neg_s)

    d = jnp.sqrt(s)
    cond = (d - min_neg_d[:, None] + MARGIN_) > 0.0
    tri = same & upper & cond

    losses = jnp.maximum(s - min_neg_s[:, None] + MARGIN_, 0.0)
    trif = tri.astype(jnp.float32)
    loss_part = jnp.sum(jnp.where(tri, losses, 0.0))
    cnt_part = jnp.sum(trif)
    acc_part = jnp.sum(jnp.where(tri & (s < min_neg_s[:, None]), 1.0, 0.0))

    @pl.when(i == 0)
    def _init():
        loss_ref[...] = jnp.zeros((1, 1), jnp.float32)
        cnt_ref[...] = jnp.zeros((1, 1), jnp.float32)
        acc_ref[...] = jnp.zeros((1, 1), jnp.float32)

    loss_ref[...] += loss_part.reshape(1, 1)
    cnt_ref[...] += cnt_part.reshape(1, 1)
    acc_ref[...] += acc_part.reshape(1, 1)


@jax.jit
def kernel(embeddings, targets):
    batch, dim = embeddings.shape
    blk = 512
    t32 = targets.astype(jnp.int32).reshape(1, batch)

    loss_sum, cnt, acc_sum = pl.pallas_call(
        functools.partial(_triplet_block, blk=blk, batch=batch),
        grid=(batch // blk,),
        in_specs=[
            pl.BlockSpec((blk, dim), lambda i: (i, 0)),
            pl.BlockSpec((batch, dim), lambda i: (0, 0)),
            pl.BlockSpec((1, batch), lambda i: (0, 0)),
        ],
        out_specs=[
            pl.BlockSpec((1, 1), lambda i: (0, 0)),
            pl.BlockSpec((1, 1), lambda i: (0, 0)),
            pl.BlockSpec((1, 1), lambda i: (0, 0)),
        ],
        out_shape=[
            jax.ShapeDtypeStruct((1, 1), jnp.float32),
            jax.ShapeDtypeStruct((1, 1), jnp.float32),
            jax.ShapeDtypeStruct((1, 1), jnp.float32),
        ],
    )(embeddings, embeddings, t32)

    loss = loss_sum[0, 0] / cnt[0, 0]
    accuracy = acc_sum[0, 0] / cnt[0, 0]
    return (loss, accuracy)


# speedup vs baseline: 2.2747x; 1.2384x over previous
import functools

import jax
import jax.numpy as jnp
from jax.experimental import pallas as pl
from jax.experimental.pallas import tpu as pltpu

MARGIN_ = 1.0


def _triplet_block(e_blk_ref, e_full_ref, t_ref, loss_ref, cnt_ref, acc_ref,
                   efs_ref, sqf_ref, *, blk: int, batch: int):
    i = pl.program_id(0)

    @pl.when(i == 0)
    def _prep():
        ef = e_full_ref[...]
        efs_ref[...] = -2.0 * ef
        sqf_ref[...] = jnp.sum(ef * ef, axis=1).reshape(1, batch)

    eb = e_blk_ref[...]
    sq_b = jnp.sum(eb * eb, axis=1)

    gram2 = jax.lax.dot_general(
        eb, efs_ref[...], (((1,), (1,)), ((), ())),
        preferred_element_type=jnp.float32)
    sp = sqf_ref[...] + gram2
    sp = jnp.maximum(sp, (-sq_b)[:, None])

    t_full = t_ref[0, :]
    t_blk = t_ref[0, pl.ds(i * blk, blk)]
    same = t_blk[:, None] == t_full[None, :]

    neg_sp = jnp.where(same, jnp.inf, sp)
    min_sp = jnp.min(neg_sp, axis=1)

    min_d = jnp.sqrt(min_sp + sq_b)
    thr = jnp.where(min_d >= MARGIN_,
                    (min_d - MARGIN_) ** 2,
                    -1.0) - sq_b
    cond = sp > thr[:, None]

    rows = i * blk + jax.lax.broadcasted_iota(jnp.int32, (blk, 1), 0)
    cols = jax.lax.broadcasted_iota(jnp.int32, (1, batch), 1)
    upper = rows < cols

    tri = same & upper & cond

    u = sp + (MARGIN_ - min_sp)[:, None]
    losses = jnp.maximum(u, 0.0)
    loss_part = jnp.sum(jnp.where(tri, losses, 0.0))
    cnt_part = jnp.sum(tri.astype(jnp.float32))
    acc_part = jnp.sum((tri & (sp < min_sp[:, None])).astype(jnp.float32))

    @pl.when(i == 0)
    def _init():
        loss_ref[...] = jnp.zeros((1, 1), jnp.float32)
        cnt_ref[...] = jnp.zeros((1, 1), jnp.float32)
        acc_ref[...] = jnp.zeros((1, 1), jnp.float32)

    loss_ref[...] += loss_part.reshape(1, 1)
    cnt_ref[...] += cnt_part.reshape(1, 1)
    acc_ref[...] += acc_part.reshape(1, 1)


@jax.jit
def kernel(embeddings, targets):
    batch, dim = embeddings.shape
    blk = 512
    t32 = targets.astype(jnp.int32).reshape(1, batch)

    loss_sum, cnt, acc_sum = pl.pallas_call(
        functools.partial(_triplet_block, blk=blk, batch=batch),
        grid=(batch // blk,),
        in_specs=[
            pl.BlockSpec((blk, dim), lambda i: (i, 0)),
            pl.BlockSpec((batch, dim), lambda i: (0, 0)),
            pl.BlockSpec((1, batch), lambda i: (0, 0)),
        ],
        out_specs=[
            pl.BlockSpec((1, 1), lambda i: (0, 0)),
            pl.BlockSpec((1, 1), lambda i: (0, 0)),
            pl.BlockSpec((1, 1), lambda i: (0, 0)),
        ],
        out_shape=[
            jax.ShapeDtypeStruct((1, 1), jnp.float32),
            jax.ShapeDtypeStruct((1, 1), jnp.float32),
            jax.ShapeDtypeStruct((1, 1), jnp.float32),
        ],
        scratch_shapes=[
            pltpu.VMEM((batch, dim), jnp.float32),
            pltpu.VMEM((1, batch), jnp.float32),
        ],
    )(embeddings, embeddings, t32)

    loss = loss_sum[0, 0] / cnt[0, 0]
    accuracy = acc_sum[0, 0] / cnt[0, 0]
    return (loss, accuracy)


# speedup vs baseline: 2.3287x; 1.0238x over previous
import functools

import jax
import jax.numpy as jnp
from jax.experimental import pallas as pl
from jax.experimental.pallas import tpu as pltpu

MARGIN_ = 1.0


def _triplet_block(e_blk_ref, e_full_ref, t_ref, loss_ref, cnt_ref, acc_ref,
                   efs_ref, sqf_ref, *, blk: int, batch: int):
    i = pl.program_id(0)

    @pl.when(i == 0)
    def _prep():
        ef = e_full_ref[...]
        efs_ref[...] = -2.0 * ef
        sqf_ref[...] = jnp.sum(ef * ef, axis=1).reshape(1, batch)

    eb = e_blk_ref[...]
    sq_b = jnp.sum(eb * eb, axis=1)

    gram2 = jax.lax.dot_general(
        eb, efs_ref[...], (((1,), (1,)), ((), ())),
        preferred_element_type=jnp.float32)
    sp = sqf_ref[...] + gram2
    sp = jnp.maximum(sp, (-sq_b)[:, None])

    t_full = t_ref[0, :]
    t_blk = t_ref[0, pl.ds(i * blk, blk)]
    same = t_blk[:, None] == t_full[None, :]

    neg_sp = jnp.where(same, jnp.inf, sp)
    min_sp = jnp.min(neg_sp, axis=1)

    min_d = jnp.sqrt(min_sp + sq_b)
    thr = jnp.where(min_d >= MARGIN_,
                    (min_d - MARGIN_) ** 2,
                    -1.0) - sq_b
    cond = sp > thr[:, None]

    rows = i * blk + jax.lax.broadcasted_iota(jnp.int32, (blk, 1), 0)
    cols = jax.lax.broadcasted_iota(jnp.int32, (1, batch), 1)
    upper = rows < cols

    tri = same & upper & cond

    u = sp + (MARGIN_ - min_sp)[:, None]
    losses = jnp.maximum(u, 0.0)
    loss_part = jnp.sum(jnp.where(tri, losses, 0.0))
    cnt_part = jnp.sum(tri.astype(jnp.float32))
    acc_part = jnp.sum((tri & (sp < min_sp[:, None])).astype(jnp.float32))

    @pl.when(i == 0)
    def _init():
        loss_ref[...] = jnp.zeros((1, 1), jnp.float32)
        cnt_ref[...] = jnp.zeros((1, 1), jnp.float32)
        acc_ref[...] = jnp.zeros((1, 1), jnp.float32)

    loss_ref[...] += loss_part.reshape(1, 1)
    cnt_ref[...] += cnt_part.reshape(1, 1)
    acc_ref[...] += acc_part.reshape(1, 1)


@jax.jit
def kernel(embeddings, targets):
    batch, dim = embeddings.shape
    blk = 1024
    t32 = targets.astype(jnp.int32).reshape(1, batch)

    loss_sum, cnt, acc_sum = pl.pallas_call(
        functools.partial(_triplet_block, blk=blk, batch=batch),
        grid=(batch // blk,),
        in_specs=[
            pl.BlockSpec((blk, dim), lambda i: (i, 0)),
            pl.BlockSpec((batch, dim), lambda i: (0, 0)),
            pl.BlockSpec((1, batch), lambda i: (0, 0)),
        ],
        out_specs=[
            pl.BlockSpec((1, 1), lambda i: (0, 0)),
            pl.BlockSpec((1, 1), lambda i: (0, 0)),
            pl.BlockSpec((1, 1), lambda i: (0, 0)),
        ],
        out_shape=[
            jax.ShapeDtypeStruct((1, 1), jnp.float32),
            jax.ShapeDtypeStruct((1, 1), jnp.float32),
            jax.ShapeDtypeStruct((1, 1), jnp.float32),
        ],
        scratch_shapes=[
            pltpu.VMEM((batch, dim), jnp.float32),
            pltpu.VMEM((1, batch), jnp.float32),
        ],
    )(embeddings, embeddings, t32)

    loss = loss_sum[0, 0] / cnt[0, 0]
    accuracy = acc_sum[0, 0] / cnt[0, 0]
    return (loss, accuracy)


# speedup vs baseline: 3.4038x; 1.4617x over previous
import functools

import jax
import jax.numpy as jnp
from jax.experimental import pallas as pl
from jax.experimental.pallas import tpu as pltpu

MARGIN_ = 1.0


def _triplet_block(e_blk_ref, e_full_ref, t_ref, loss_ref, cnt_ref, acc_ref,
                   efs_ref, sqf_ref, *, blk: int, batch: int, cblk: int):
    i = pl.program_id(0)

    @pl.when(i == 0)
    def _prep():
        ef = e_full_ref[...]
        efs_ref[...] = -2.0 * ef
        sqf_ref[...] = jnp.sum(ef * ef, axis=1).reshape(1, batch)

    eb = e_blk_ref[...]
    sq_b = jnp.sum(eb * eb, axis=1)

    gram2 = jax.lax.dot_general(
        eb, efs_ref[...], (((1,), (1,)), ((), ())),
        preferred_element_type=jnp.float32)
    sp = sqf_ref[...] + gram2
    sp = jnp.maximum(sp, (-sq_b)[:, None])

    t_full = t_ref[0, :]
    t_blk = t_ref[0, pl.ds(i * blk, blk)]
    same = t_blk[:, None] == t_full[None, :]

    neg_sp = jnp.where(same, jnp.inf, sp)
    min_sp = jnp.min(neg_sp, axis=1)

    min_d = jnp.sqrt(min_sp + sq_b)
    thr = (jnp.where(min_d >= MARGIN_,
                     (min_d - MARGIN_) ** 2,
                     -1.0) - sq_b)[:, None]
    ushift = (MARGIN_ - min_sp)[:, None]
    minsp_col = min_sp[:, None]

    @pl.when(i == 0)
    def _init():
        loss_ref[...] = jnp.zeros((1, 1), jnp.float32)
        cnt_ref[...] = jnp.zeros((1, 1), jnp.float32)
        acc_ref[...] = jnp.zeros((1, 1), jnp.float32)

    ratio = blk // cblk
    rows = i * blk + jax.lax.broadcasted_iota(jnp.int32, (blk, 1), 0)

    def chunk_sums(c, need_upper):
        sl = slice(c * cblk, (c + 1) * cblk)
        spc = sp[:, sl]
        tri = same[:, sl] & (spc > thr)
        if need_upper:
            cols = c * cblk + jax.lax.broadcasted_iota(jnp.int32, (1, cblk), 1)
            tri = tri & (rows < cols)
        losses = jnp.maximum(spc + ushift, 0.0)
        loss_part = jnp.sum(jnp.where(tri, losses, 0.0))
        cnt_part = jnp.sum(tri.astype(jnp.float32))
        acc_part = jnp.sum((tri & (spc < minsp_col)).astype(jnp.float32))
        loss_ref[...] += loss_part.reshape(1, 1)
        cnt_ref[...] += cnt_part.reshape(1, 1)
        acc_ref[...] += acc_part.reshape(1, 1)

    for c in range(batch // cblk):
        @pl.when(c >= (i + 1) * ratio)
        def _full(c=c):
            chunk_sums(c, need_upper=False)

        @pl.when((c >= i * ratio) & (c < (i + 1) * ratio))
        def _diag(c=c):
            chunk_sums(c, need_upper=True)


@jax.jit
def kernel(embeddings, targets):
    batch, dim = embeddings.shape
    blk = 1024
    cblk = 512
    t32 = targets.astype(jnp.int32).reshape(1, batch)

    loss_sum, cnt, acc_sum = pl.pallas_call(
        functools.partial(_triplet_block, blk=blk, batch=batch, cblk=cblk),
        grid=(batch // blk,),
        in_specs=[
            pl.BlockSpec((blk, dim), lambda i: (i, 0)),
            pl.BlockSpec((batch, dim), lambda i: (0, 0)),
            pl.BlockSpec((1, batch), lambda i: (0, 0)),
        ],
        out_specs=[
            pl.BlockSpec((1, 1), lambda i: (0, 0)),
            pl.BlockSpec((1, 1), lambda i: (0, 0)),
            pl.BlockSpec((1, 1), lambda i: (0, 0)),
        ],
        out_shape=[
            jax.ShapeDtypeStruct((1, 1), jnp.float32),
            jax.ShapeDtypeStruct((1, 1), jnp.float32),
            jax.ShapeDtypeStruct((1, 1), jnp.float32),
        ],
        scratch_shapes=[
            pltpu.VMEM((batch, dim), jnp.float32),
            pltpu.VMEM((1, batch), jnp.float32),
        ],
    )(embeddings, embeddings, t32)

    loss = loss_sum[0, 0] / cnt[0, 0]
    accuracy = acc_sum[0, 0] / cnt[0, 0]
    return (loss, accuracy)


# speedup vs baseline: 3.4721x; 1.0201x over previous
import functools

import jax
import jax.numpy as jnp
from jax.experimental import pallas as pl
from jax.experimental.pallas import tpu as pltpu

MARGIN_ = 1.0


def _triplet_block(e_blk_ref, e_full_ref, t_ref, loss_ref, cnt_ref, acc_ref,
                   efs_ref, sqf_ref, lvec_ref, cvec_ref, avec_ref,
                   *, blk: int, batch: int, cblk: int):
    i = pl.program_id(0)

    @pl.when(i == 0)
    def _prep():
        ef = e_full_ref[...]
        efs_ref[...] = -2.0 * ef
        sqf_ref[...] = jnp.sum(ef * ef, axis=1).reshape(1, batch)

    eb = e_blk_ref[...]
    sq_b = jnp.sum(eb * eb, axis=1)

    gram2 = jax.lax.dot_general(
        eb, efs_ref[...], (((1,), (1,)), ((), ())),
        preferred_element_type=jnp.float32)
    sp = sqf_ref[...] + gram2
    sp = jnp.maximum(sp, (-sq_b)[:, None])

    t_full = t_ref[0, :]
    t_blk = t_ref[0, pl.ds(i * blk, blk)]
    same = t_blk[:, None] == t_full[None, :]

    neg_sp = jnp.where(same, jnp.inf, sp)
    min_sp = jnp.min(neg_sp, axis=1)

    min_d = jnp.sqrt(min_sp + sq_b)
    thr = (jnp.where(min_d >= MARGIN_,
                     (min_d - MARGIN_) ** 2,
                     -1.0) - sq_b)[:, None]
    ushift = (MARGIN_ - min_sp)[:, None]
    minsp_col = min_sp[:, None]

    @pl.when(i == 0)
    def _init():
        lvec_ref[...] = jnp.zeros((1, cblk), jnp.float32)
        cvec_ref[...] = jnp.zeros((1, cblk), jnp.float32)
        avec_ref[...] = jnp.zeros((1, cblk), jnp.float32)

    ratio = blk // cblk
    rows = i * blk + jax.lax.broadcasted_iota(jnp.int32, (blk, 1), 0)
    ones_row = jnp.ones((1, blk), jnp.float32)

    def chunk_sums(c, need_upper):
        sl = slice(c * cblk, (c + 1) * cblk)
        spc = sp[:, sl]
        tri = same[:, sl] & (spc > thr)
        if need_upper:
            cols = c * cblk + jax.lax.broadcasted_iota(jnp.int32, (1, cblk), 1)
            tri = tri & (rows < cols)
        trif = tri.astype(jnp.float32)
        lossm = trif * jnp.maximum(spc + ushift, 0.0)
        accm = jnp.where(spc < minsp_col, trif, 0.0)
        dot = lambda x: jax.lax.dot_general(
            ones_row, x, (((1,), (0,)), ((), ())),
            preferred_element_type=jnp.float32)
        lvec_ref[...] += dot(lossm)
        cvec_ref[...] += dot(trif)
        avec_ref[...] += dot(accm)

    for c in range(batch // cblk):
        @pl.when(c >= (i + 1) * ratio)
        def _full(c=c):
            chunk_sums(c, need_upper=False)

        @pl.when((c >= i * ratio) & (c < (i + 1) * ratio))
        def _diag(c=c):
            chunk_sums(c, need_upper=True)

    @pl.when(i == pl.num_programs(0) - 1)
    def _finish():
        loss_ref[...] = jnp.sum(lvec_ref[...]).reshape(1, 1)
        cnt_ref[...] = jnp.sum(cvec_ref[...]).reshape(1, 1)
        acc_ref[...] = jnp.sum(avec_ref[...]).reshape(1, 1)


@jax.jit
def kernel(embeddings, targets):
    batch, dim = embeddings.shape
    blk = 1024
    cblk = 512
    t32 = targets.astype(jnp.int32).reshape(1, batch)

    loss_sum, cnt, acc_sum = pl.pallas_call(
        functools.partial(_triplet_block, blk=blk, batch=batch, cblk=cblk),
        grid=(batch // blk,),
        in_specs=[
            pl.BlockSpec((blk, dim), lambda i: (i, 0)),
            pl.BlockSpec((batch, dim), lambda i: (0, 0)),
            pl.BlockSpec((1, batch), lambda i: (0, 0)),
        ],
        out_specs=[
            pl.BlockSpec((1, 1), lambda i: (0, 0)),
            pl.BlockSpec((1, 1), lambda i: (0, 0)),
            pl.BlockSpec((1, 1), lambda i: (0, 0)),
        ],
        out_shape=[
            jax.ShapeDtypeStruct((1, 1), jnp.float32),
            jax.ShapeDtypeStruct((1, 1), jnp.float32),
            jax.ShapeDtypeStruct((1, 1), jnp.float32),
        ],
        scratch_shapes=[
            pltpu.VMEM((batch, dim), jnp.float32),
            pltpu.VMEM((1, batch), jnp.float32),
            pltpu.VMEM((1, cblk), jnp.float32),
            pltpu.VMEM((1, cblk), jnp.float32),
            pltpu.VMEM((1, cblk), jnp.float32),
        ],
    )(embeddings, embeddings, t32)

    loss = loss_sum[0, 0] / cnt[0, 0]
    accuracy = acc_sum[0, 0] / cnt[0, 0]
    return (loss, accuracy)


# speedup vs baseline: 3.9036x; 1.1243x over previous
import functools

import jax
import jax.numpy as jnp
from jax.experimental import pallas as pl
from jax.experimental.pallas import tpu as pltpu

MARGIN_ = 1.0


def _triplet_block(e_blk_ref, e_full_ref, t_ref, loss_ref, cnt_ref, acc_ref,
                   efa_ref, eba_ref, lvec_ref, cvec_ref, avec_ref,
                   *, blk: int, batch: int, cblk: int, dim: int):
    i = pl.program_id(0)
    ka = dim + 8

    @pl.when(i == 0)
    def _prep():
        ef = e_full_ref[...]
        efa_ref[:, pl.ds(0, dim)] = -2.0 * ef
        efa_ref[:, pl.ds(dim, 8)] = jnp.concatenate(
            [jnp.sum(ef * ef, axis=1, keepdims=True),
             jnp.zeros((batch, 7), jnp.float32)], axis=1)
        eba_ref[:, pl.ds(dim, 8)] = jnp.concatenate(
            [jnp.ones((blk, 1), jnp.float32),
             jnp.zeros((blk, 7), jnp.float32)], axis=1)

    eb = e_blk_ref[...]
    sq_b = jnp.sum(eb * eb, axis=1)
    eba_ref[:, pl.ds(0, dim)] = eb

    sp = jax.lax.dot_general(
        eba_ref[...], efa_ref[...], (((1,), (1,)), ((), ())),
        preferred_element_type=jnp.float32)

    t_full = t_ref[0, :]
    t_blk = t_ref[0, pl.ds(i * blk, blk)]
    same = t_blk[:, None] == t_full[None, :]

    neg_sp = jnp.where(same, jnp.inf, sp)
    min_sp = jnp.maximum(jnp.min(neg_sp, axis=1), -sq_b)

    min_d = jnp.sqrt(jnp.maximum(min_sp + sq_b, 0.0))
    thr = (jnp.where(min_d >= MARGIN_,
                     (min_d - MARGIN_) ** 2,
                     -1.0) - sq_b)[:, None]
    ushift = (MARGIN_ - min_sp)[:, None]
    minsp_col = min_sp[:, None]

    @pl.when(i == 0)
    def _init():
        lvec_ref[...] = jnp.zeros((1, cblk), jnp.float32)
        cvec_ref[...] = jnp.zeros((1, cblk), jnp.float32)
        avec_ref[...] = jnp.zeros((1, cblk), jnp.float32)

    ratio = blk // cblk
    rows = i * blk + jax.lax.broadcasted_iota(jnp.int32, (blk, 1), 0)
    ones_row = jnp.ones((1, blk), jnp.float32)

    def chunk_sums(c, need_upper):
        sl = slice(c * cblk, (c + 1) * cblk)
        spc = sp[:, sl]
        tri = same[:, sl] & (spc > thr)
        if need_upper:
            cols = c * cblk + jax.lax.broadcasted_iota(jnp.int32, (1, cblk), 1)
            tri = tri & (rows < cols)
        trif = tri.astype(jnp.float32)
        lossm = trif * jnp.maximum(spc + ushift, 0.0)
        accm = jnp.where(spc < minsp_col, trif, 0.0)
        dot = lambda x: jax.lax.dot_general(
            ones_row, x, (((1,), (0,)), ((), ())),
            preferred_element_type=jnp.float32)
        lvec_ref[...] += dot(lossm)
        cvec_ref[...] += dot(trif)
        avec_ref[...] += dot(accm)

    for c in range(batch // cblk):
        @pl.when(c >= (i + 1) * ratio)
        def _full(c=c):
            chunk_sums(c, need_upper=False)

        @pl.when((c >= i * ratio) & (c < (i + 1) * ratio))
        def _diag(c=c):
            chunk_sums(c, need_upper=True)

    @pl.when(i == pl.num_programs(0) - 1)
    def _finish():
        loss_ref[...] = jnp.sum(lvec_ref[...]).reshape(1, 1)
        cnt_ref[...] = jnp.sum(cvec_ref[...]).reshape(1, 1)
        acc_ref[...] = jnp.sum(avec_ref[...]).reshape(1, 1)


@jax.jit
def kernel(embeddings, targets):
    batch, dim = embeddings.shape
    blk = 1024
    cblk = 512
    t32 = targets.astype(jnp.int32).reshape(1, batch)

    loss_sum, cnt, acc_sum = pl.pallas_call(
        functools.partial(_triplet_block, blk=blk, batch=batch, cblk=cblk,
                          dim=dim),
        grid=(batch // blk,),
        in_specs=[
            pl.BlockSpec((blk, dim), lambda i: (i, 0)),
            pl.BlockSpec((batch, dim), lambda i: (0, 0)),
            pl.BlockSpec((1, batch), lambda i: (0, 0)),
        ],
        out_specs=[
            pl.BlockSpec((1, 1), lambda i: (0, 0)),
            pl.BlockSpec((1, 1), lambda i: (0, 0)),
            pl.BlockSpec((1, 1), lambda i: (0, 0)),
        ],
        out_shape=[
            jax.ShapeDtypeStruct((1, 1), jnp.float32),
            jax.ShapeDtypeStruct((1, 1), jnp.float32),
            jax.ShapeDtypeStruct((1, 1), jnp.float32),
        ],
        scratch_shapes=[
            pltpu.VMEM((batch, dim + 8), jnp.float32),
            pltpu.VMEM((blk, dim + 8), jnp.float32),
            pltpu.VMEM((1, cblk), jnp.float32),
            pltpu.VMEM((1, cblk), jnp.float32),
            pltpu.VMEM((1, cblk), jnp.float32),
        ],
    )(embeddings, embeddings, t32)

    loss = loss_sum[0, 0] / cnt[0, 0]
    accuracy = acc_sum[0, 0] / cnt[0, 0]
    return (loss, accuracy)
